# baseline (speedup 1.0000x reference)
import functools

import jax
import jax.numpy as jnp
from jax import lax
from jax.experimental import pallas as pl
from jax.experimental.pallas import tpu as pltpu
from jax.experimental.pallas import tpu_sc as plsc

N = 10000
E = 320000
D = 128
H = 2 * D
EPS = 1e-7
BN_EPS = 1e-5

NCORE = 2
NSUB = 16
NW = NCORE * NSUB
CPT = D // NW
CHE = 1600
NCHE = E // CHE
WPC = CHE // 16


def _sc_agg_body(xt_hbm, src_hbm, dst_hbm, out_hbm, xch,
                 a0, a1, a2, a3, a4, a5, a6, a7,
                 sb0, sb1, db0, db1, isem0, isem1):
    c = lax.axis_index("c")
    s = lax.axis_index("s")
    t = s * NCORE + c
    accs = (a0, a1, a2, a3, a4, a5, a6, a7)

    pltpu.sync_copy(xt_hbm.at[t], xch)

    zeros = jnp.zeros((16,), jnp.float32)

    def zrow(i, carry):
        for a in accs:
            a[pl.ds(i * 16, 16)] = zeros
        return carry

    lax.fori_loop(0, N // 16, zrow, 0)

    def window(sbuf, dbuf, wi):
        sv = sbuf[pl.ds(wi * 16, 16)]
        dv = dbuf[pl.ds(wi * 16, 16)]
        cnt, _ = plsc.scan_count(dv)
        rmin = jnp.min(cnt)
        rmax = jnp.max(cnt)
        m0 = cnt == rmin
        ws = []
        mws = []
        for ch in range(CPT):
            gi = sv + (ch * N) if ch else sv
            v = plsc.load_gather(xch, [gi])
            m = jnp.maximum(v, 0.0) + EPS
            w = jnp.exp(m)
            mw = m * w
            ws.append(w)
            mws.append(mw)
            plsc.addupdate_scatter(accs[2 * ch], [dv], w, mask=m0)
            plsc.addupdate_scatter(accs[2 * ch + 1], [dv], mw, mask=m0)

        @pl.when(rmax > rmin)
        def _():
            def rnd(r, carry):
                mr = cnt == (rmin + r)
                for ch in range(CPT):
                    plsc.addupdate_scatter(accs[2 * ch], [dv], ws[ch],
                                           mask=mr)
                    plsc.addupdate_scatter(accs[2 * ch + 1], [dv], mws[ch],
                                           mask=mr)
                return carry

            lax.fori_loop(1, rmax - rmin + 1, rnd, 0)

    def do_chunk(ci, bufs, nbufs):
        sbuf, dbuf, sem = bufs
        nsbuf, ndbuf, nsem = nbufs
        pltpu.make_async_copy(src_hbm.at[pl.ds(ci * CHE, CHE)], sbuf,
                              sem).wait()
        pltpu.make_async_copy(dst_hbm.at[pl.ds(ci * CHE, CHE)], dbuf,
                              sem).wait()

        @pl.when(ci + 1 < NCHE)
        def _():
            nci = ci + 1
            pltpu.async_copy(src_hbm.at[pl.ds(nci * CHE, CHE)], nsbuf, nsem)
            pltpu.async_copy(dst_hbm.at[pl.ds(nci * CHE, CHE)], ndbuf, nsem)

        def wbody(wi, carry):
            window(sbuf, dbuf, wi)
            return carry

        lax.fori_loop(0, WPC, wbody, 0)

    bufs0 = (sb0, db0, isem0)
    bufs1 = (sb1, db1, isem1)
    pltpu.async_copy(src_hbm.at[pl.ds(0, CHE)], sb0, isem0)
    pltpu.async_copy(dst_hbm.at[pl.ds(0, CHE)], db0, isem0)

    def pair(i, carry):
        do_chunk(2 * i, bufs0, bufs1)
        do_chunk(2 * i + 1, bufs1, bufs0)
        return carry

    lax.fori_loop(0, NCHE // 2, pair, 0)

    for j, a in enumerate(accs):
        pltpu.sync_copy(a, out_hbm.at[t, j])


_sc_agg = functools.partial(
    pl.kernel,
    out_type=jax.ShapeDtypeStruct((NW, 2 * CPT, N), jnp.float32),
    mesh=plsc.VectorSubcoreMesh(core_axis_name="c", subcore_axis_name="s",
                                num_cores=NCORE),
    compiler_params=pltpu.CompilerParams(needs_layout_passes=False),
    scratch_types=[
        pltpu.VMEM((CPT * N,), jnp.float32),
        pltpu.VMEM((N,), jnp.float32),
        pltpu.VMEM((N,), jnp.float32),
        pltpu.VMEM((N,), jnp.float32),
        pltpu.VMEM((N,), jnp.float32),
        pltpu.VMEM((N,), jnp.float32),
        pltpu.VMEM((N,), jnp.float32),
        pltpu.VMEM((N,), jnp.float32),
        pltpu.VMEM((N,), jnp.float32),
        pltpu.VMEM((CHE,), jnp.int32),
        pltpu.VMEM((CHE,), jnp.int32),
        pltpu.VMEM((CHE,), jnp.int32),
        pltpu.VMEM((CHE,), jnp.int32),
        pltpu.SemaphoreType.DMA,
        pltpu.SemaphoreType.DMA,
    ],
)(_sc_agg_body)


def _tc_body(x_ref, den_ref, num_ref, w1_ref, b1_ref, g_ref, be_ref, w2_ref,
             b2_ref, o_ref):
    x = x_ref[...]
    out = num_ref[...] / (den_ref[...] + 1e-16) + x
    h = jnp.dot(out, w1_ref[...], preferred_element_type=jnp.float32)
    h = h + b1_ref[...]
    mean = jnp.mean(h, axis=0, keepdims=True)
    var = jnp.mean((h - mean) ** 2, axis=0, keepdims=True)
    hn = (h - mean) * lax.rsqrt(var + BN_EPS) * g_ref[...] + be_ref[...]
    hn = jnp.maximum(hn, 0.0)
    y = jnp.dot(hn, w2_ref[...], preferred_element_type=jnp.float32)
    y = y + b2_ref[...]
    o_ref[...] = x + jnp.maximum(y, 0.0)


def kernel(x, edge_index, W1, b1, gamma, beta, W2, b2):
    ei = edge_index.astype(jnp.int32)
    xt = jnp.transpose(x).reshape(NW, CPT * N)
    o = _sc_agg(xt, ei[0], ei[1])
    den = jnp.transpose(o[:, 0::2, :].reshape(D, N))
    num = jnp.transpose(o[:, 1::2, :].reshape(D, N))
    return pl.pallas_call(
        _tc_body,
        out_shape=jax.ShapeDtypeStruct((N, D), jnp.float32),
    )(x, den, num, W1, b1[None, :], gamma[None, :], beta[None, :], W2,
      b2[None, :])


# speedup vs baseline: 1.2401x; 1.2401x over previous
import functools

import jax
import jax.numpy as jnp
from jax import lax
from jax.experimental import pallas as pl
from jax.experimental.pallas import tpu as pltpu
from jax.experimental.pallas import tpu_sc as plsc

N = 10000
E = 320000
D = 128
H = 2 * D
EPS = 1e-7
BN_EPS = 1e-5

NCORE = 2
NSUB = 16
NW = NCORE * NSUB
CPT = D // NW
CHE = 1600
NCHE = E // CHE
WPC = CHE // 16


def _sc_agg_body(xt_hbm, src_hbm, dst_hbm, out_hbm, xch,
                 a0, a1, a2, a3, a4, a5, a6, a7,
                 sb0, sb1, db0, db1, isem0, isem1):
    c = lax.axis_index("c")
    s = lax.axis_index("s")
    t = s * NCORE + c
    accs = (a0, a1, a2, a3, a4, a5, a6, a7)

    pltpu.sync_copy(xt_hbm.at[t], xch)

    zeros = jnp.zeros((16,), jnp.float32)

    def zrow(i, carry):
        for a in accs:
            a[pl.ds(i * 16, 16)] = zeros
        return carry

    lax.fori_loop(0, N // 16, zrow, 0)

    def window(sbuf, dbuf, wi):
        sv = sbuf[pl.ds(wi * 16, 16)]
        dv = dbuf[pl.ds(wi * 16, 16)]
        for ch in range(CPT):
            gi = sv + (ch * N) if ch else sv
            v = plsc.load_gather(xch, [gi])
            m = jnp.maximum(v, 0.0) + EPS
            w = jnp.exp(m)
            mw = m * w
            plsc.addupdate_scatter(accs[2 * ch], [dv], w)
            plsc.addupdate_scatter(accs[2 * ch + 1], [dv], mw)

    def do_chunk(ci, bufs, nbufs):
        sbuf, dbuf, sem = bufs
        nsbuf, ndbuf, nsem = nbufs
        pltpu.make_async_copy(src_hbm.at[pl.ds(ci * CHE, CHE)], sbuf,
                              sem).wait()
        pltpu.make_async_copy(dst_hbm.at[pl.ds(ci * CHE, CHE)], dbuf,
                              sem).wait()

        @pl.when(ci + 1 < NCHE)
        def _():
            nci = ci + 1
            pltpu.async_copy(src_hbm.at[pl.ds(nci * CHE, CHE)], nsbuf, nsem)
            pltpu.async_copy(dst_hbm.at[pl.ds(nci * CHE, CHE)], ndbuf, nsem)

        def wbody(wi, carry):
            window(sbuf, dbuf, wi)
            return carry

        lax.fori_loop(0, WPC, wbody, 0)

    bufs0 = (sb0, db0, isem0)
    bufs1 = (sb1, db1, isem1)
    pltpu.async_copy(src_hbm.at[pl.ds(0, CHE)], sb0, isem0)
    pltpu.async_copy(dst_hbm.at[pl.ds(0, CHE)], db0, isem0)

    def pair(i, carry):
        do_chunk(2 * i, bufs0, bufs1)
        do_chunk(2 * i + 1, bufs1, bufs0)
        return carry

    lax.fori_loop(0, NCHE // 2, pair, 0)

    for j, a in enumerate(accs):
        pltpu.sync_copy(a, out_hbm.at[t, j])


_sc_agg = functools.partial(
    pl.kernel,
    out_type=jax.ShapeDtypeStruct((NW, 2 * CPT, N), jnp.float32),
    mesh=plsc.VectorSubcoreMesh(core_axis_name="c", subcore_axis_name="s",
                                num_cores=NCORE),
    compiler_params=pltpu.CompilerParams(needs_layout_passes=False),
    scratch_types=[
        pltpu.VMEM((CPT * N,), jnp.float32),
        pltpu.VMEM((N,), jnp.float32),
        pltpu.VMEM((N,), jnp.float32),
        pltpu.VMEM((N,), jnp.float32),
        pltpu.VMEM((N,), jnp.float32),
        pltpu.VMEM((N,), jnp.float32),
        pltpu.VMEM((N,), jnp.float32),
        pltpu.VMEM((N,), jnp.float32),
        pltpu.VMEM((N,), jnp.float32),
        pltpu.VMEM((CHE,), jnp.int32),
        pltpu.VMEM((CHE,), jnp.int32),
        pltpu.VMEM((CHE,), jnp.int32),
        pltpu.VMEM((CHE,), jnp.int32),
        pltpu.SemaphoreType.DMA,
        pltpu.SemaphoreType.DMA,
    ],
)(_sc_agg_body)


def _tc_body(x_ref, den_ref, num_ref, w1_ref, b1_ref, g_ref, be_ref, w2_ref,
             b2_ref, o_ref):
    x = x_ref[...]
    out = num_ref[...] / (den_ref[...] + 1e-16) + x
    h = jnp.dot(out, w1_ref[...], preferred_element_type=jnp.float32)
    h = h + b1_ref[...]
    mean = jnp.mean(h, axis=0, keepdims=True)
    var = jnp.mean((h - mean) ** 2, axis=0, keepdims=True)
    hn = (h - mean) * lax.rsqrt(var + BN_EPS) * g_ref[...] + be_ref[...]
    hn = jnp.maximum(hn, 0.0)
    y = jnp.dot(hn, w2_ref[...], preferred_element_type=jnp.float32)
    y = y + b2_ref[...]
    o_ref[...] = x + jnp.maximum(y, 0.0)


def kernel(x, edge_index, W1, b1, gamma, beta, W2, b2):
    ei = edge_index.astype(jnp.int32)
    xt = jnp.transpose(x).reshape(NW, CPT * N)
    o = _sc_agg(xt, ei[0], ei[1])
    den = jnp.transpose(o[:, 0::2, :].reshape(D, N))
    num = jnp.transpose(o[:, 1::2, :].reshape(D, N))
    return pl.pallas_call(
        _tc_body,
        out_shape=jax.ShapeDtypeStruct((N, D), jnp.float32),
    )(x, den, num, W1, b1[None, :], gamma[None, :], beta[None, :], W2,
      b2[None, :])
